# Initial kernel scaffold; baseline (speedup 1.0000x reference)
import jax
import jax.numpy as jnp
from jax.experimental import pallas as pl


def kernel(points, features, cls_preds, W1, b1, gamma, beta, W2, max_offset_limit):
    raise NotImplementedError("write your pallas kernel here")


# speedup vs baseline: 2.9283x; 2.9283x over previous
import functools

import jax
import jax.numpy as jnp
from jax import lax
from jax.experimental import pallas as pl
from jax.experimental.pallas import tpu as pltpu
from jax.experimental.pallas import tpu_sc as plsc

B, N, K = 8, 16384, 4096
C_IN, C_MID = 128, 128
NUM_CLS = 3
BN_EPS = 1e-5

BLK = 2048

SC_CORES, SC_SUBCORES, L = 2, 16, 16
NVREG = N // L


def _tc_body(cls_ref, f_ref, pts_ref, w1_ref, b1_ref, g_ref, be_ref, w2_ref,
             mol_ref, keys_ref, predb_ref, off_ref):
  s = jnp.max(cls_ref[0], axis=0)
  s = jnp.where(s == 0.0, 0.0, s)
  u = lax.bitcast_convert_type(s, jnp.int32)
  k = jnp.where(s < 0.0, u, jnp.int32(0x7FFFFFFF) - u)
  keys_ref[0, 0] = k

  inv = 1.0 / (1.0 + BN_EPS) ** 0.5
  scale = g_ref[...] * inv
  w1e = w1_ref[...] * scale
  bias = b1_ref[...] * scale + be_ref[...]
  f = f_ref[0]
  h = lax.dot_general(w1e, f, (((1,), (0,)), ((), ())),
                      preferred_element_type=jnp.float32) + bias
  h = jnp.maximum(h, 0.0)
  off = lax.dot_general(w2_ref[...], h, (((1,), (0,)), ((), ())),
                        preferred_element_type=jnp.float32)
  mol = mol_ref[...]
  lim = jnp.where(off > mol, mol, off)
  lim = jnp.where(lim < -mol, -mol, lim)
  off_ref[0] = off
  predb_ref[0] = pts_ref[0] + lim


def _tc_stage(cls_t, features, pts_t, w1, b1_c, g_c, be_c, w2, mol_c):
  grid = (B, N // BLK)
  full = lambda b, n: (0, 0)
  return pl.pallas_call(
      _tc_body,
      grid=grid,
      in_specs=[
          pl.BlockSpec((1, NUM_CLS, BLK), lambda b, n: (b, 0, n)),
          pl.BlockSpec((1, C_IN, BLK), lambda b, n: (b, 0, n)),
          pl.BlockSpec((1, 3, BLK), lambda b, n: (b, 0, n)),
          pl.BlockSpec((C_MID, C_IN), full),
          pl.BlockSpec((C_MID, 1), full),
          pl.BlockSpec((C_MID, 1), full),
          pl.BlockSpec((C_MID, 1), full),
          pl.BlockSpec((3, C_MID), full),
          pl.BlockSpec((3, 1), full),
      ],
      out_specs=[
          pl.BlockSpec((1, 1, BLK), lambda b, n: (b, 0, n)),
          pl.BlockSpec((1, 3, BLK), lambda b, n: (b, 0, n)),
          pl.BlockSpec((1, 3, BLK), lambda b, n: (b, 0, n)),
      ],
      out_shape=[
          jax.ShapeDtypeStruct((B, 1, N), jnp.int32),
          jax.ShapeDtypeStruct((B, 3, N), jnp.float32),
          jax.ShapeDtypeStruct((B, 3, N), jnp.float32),
      ],
  )(cls_t, features, pts_t, w1, b1_c, g_c, be_c, w2, mol_c)


def _sc_body(keys_hbm, predb_hbm, off_hbm, pts_hbm,
             preds_o, orig_o, offs_o,
             keys0, keys1, idx0, idx1, hist, row, outb):
  wid = lax.axis_index("s") * SC_CORES + lax.axis_index("c")

  @pl.when(wid < B)
  def _():
    b = wid
    pltpu.sync_copy(keys_hbm.at[pl.ds(b * N, N)], keys0)

    def init_iota(j, _):
      idx0[pl.ds(j * L, L)] = lax.iota(jnp.int32, L) + j * L
      return 0
    lax.fori_loop(0, NVREG, init_iota, 0)

    for p in range(4):
      src_k, src_i = (keys0, idx0) if p % 2 == 0 else (keys1, idx1)
      dst_k, dst_i = (keys1, idx1) if p % 2 == 0 else (keys0, idx0)
      shift = 8 * p

      def zero_hist(i, _):
        hist[pl.ds(i * L, L)] = jnp.zeros((L,), jnp.int32)
        return 0
      lax.fori_loop(0, 256 // L, zero_hist, 0)

      def count(j, _, src_k=src_k, shift=shift):
        k = src_k[pl.ds(j * L, L)]
        d = lax.shift_right_logical(k, shift) & 255
        cnt, lastm = plsc.scan_count(d)
        plsc.addupdate_scatter(hist, [d], cnt, mask=lastm)
        return 0
      lax.fori_loop(0, NVREG, count, 0)

      def excl_scan(i, carry):
        chunk = hist[pl.ds(i * L, L)]
        incl = plsc.cumsum(chunk)
        hist[pl.ds(i * L, L)] = incl - chunk + carry
        return carry + jnp.max(incl)
      lax.fori_loop(0, 256 // L, excl_scan, jnp.int32(0))

      def permute(j, _, src_k=src_k, src_i=src_i, dst_k=dst_k, dst_i=dst_i,
                  shift=shift):
        k = src_k[pl.ds(j * L, L)]
        iv = src_i[pl.ds(j * L, L)]
        d = lax.shift_right_logical(k, shift) & 255
        cnt, lastm = plsc.scan_count(d)
        base = plsc.load_gather(hist, [d])
        pos = base + cnt - 1
        plsc.store_scatter(dst_k, [pos], k)
        plsc.store_scatter(dst_i, [pos], iv)
        plsc.addupdate_scatter(hist, [d], cnt, mask=lastm)
        return 0
      lax.fori_loop(0, NVREG, permute, 0)

    for src_hbm, out_hbm in ((predb_hbm, preds_o), (pts_hbm, orig_o),
                             (off_hbm, offs_o)):
      for c in range(3):
        pltpu.sync_copy(src_hbm.at[pl.ds((b * 3 + c) * N, N)], row)

        def gather(j, _):
          iv = idx0[pl.ds(j * L, L)]
          outb[pl.ds(j * L, L)] = plsc.load_gather(row, [iv])
          return 0
        lax.fori_loop(0, K // L, gather, 0)
        pltpu.sync_copy(outb, out_hbm.at[pl.ds((b * 3 + c) * K, K)])


def _sc_stage(keys, predb_t, off_t, pts_t):
  mesh = plsc.VectorSubcoreMesh(core_axis_name="c", subcore_axis_name="s",
                                num_cores=SC_CORES, num_subcores=SC_SUBCORES)
  out3k = jax.ShapeDtypeStruct((B * 3 * K,), jnp.float32)
  fn = pl.kernel(
      _sc_body,
      out_type=(out3k, out3k, out3k),
      mesh=mesh,
      compiler_params=pltpu.CompilerParams(needs_layout_passes=False),
      scratch_types=[
          pltpu.VMEM((N,), jnp.int32),
          pltpu.VMEM((N,), jnp.int32),
          pltpu.VMEM((N,), jnp.int32),
          pltpu.VMEM((N,), jnp.int32),
          pltpu.VMEM((256,), jnp.int32),
          pltpu.VMEM((N,), jnp.float32),
          pltpu.VMEM((K,), jnp.float32),
      ],
  )
  p, o, f = fn(keys.reshape(-1), predb_t.reshape(-1), off_t.reshape(-1),
               pts_t.reshape(-1))
  return (p.reshape(B, 3, K), o.reshape(B, 3, K), f.reshape(B, 3, K))


def kernel(points, features, cls_preds, W1, b1, gamma, beta, W2,
           max_offset_limit):
  cls_t = jnp.transpose(cls_preds, (0, 2, 1))
  pts_t = jnp.transpose(points, (0, 2, 1))
  keys, predb_t, off_t = _tc_stage(
      cls_t, features, pts_t, W1,
      b1.reshape(C_MID, 1), gamma.reshape(C_MID, 1), beta.reshape(C_MID, 1),
      W2, max_offset_limit.reshape(3, 1))
  preds_t, orig_t, offs_t = _sc_stage(keys, predb_t, off_t, pts_t)
  return (jnp.transpose(preds_t, (0, 2, 1)),
          jnp.transpose(orig_t, (0, 2, 1)),
          jnp.transpose(offs_t, (0, 2, 1)))
